# baseline (speedup 1.0000x reference)
import functools

import jax
import jax.numpy as jnp
from jax import lax
from jax.experimental import pallas as pl
from jax.experimental.pallas import tpu as pltpu
from jax.experimental.pallas import tpu_sc as plsc

NC, NS = 2, 16
NW = NC * NS
C = 4
R = 10240


def _gather_sc(idx_flat, table, base_row, rows):
    d = table.shape[1]
    per_w = rows // NW
    ch = next(c for c in range(min(per_w, 1280), 0, -1)
              if per_w % c == 0 and c % 8 == 0)
    n_ch = per_w // ch
    mesh = plsc.VectorSubcoreMesh(core_axis_name="c", subcore_axis_name="s")

    @functools.partial(
        pl.kernel,
        mesh=mesh,
        out_type=jax.ShapeDtypeStruct((rows, d), jnp.float32),
        scratch_types=[
            pltpu.VMEM((ch,), jnp.int32),
            pltpu.VMEM((ch, d), jnp.float32),
            pltpu.SemaphoreType.DMA,
        ],
        compiler_params=pltpu.CompilerParams(use_tc_tiling_on_sc=False),
    )
    def gather_kernel(idx_hbm, table_hbm, out_hbm, idx_v, rows_v, sem):
        wid = lax.axis_index("s") * NC + lax.axis_index("c")
        base = wid * per_w

        def body(i, carry):
            off = base + i * ch
            pltpu.sync_copy(idx_hbm.at[pl.ds(base_row + off, ch)], idx_v)
            pltpu.async_copy(table_hbm.at[idx_v], rows_v, sem).wait()
            pltpu.sync_copy(rows_v, out_hbm.at[pl.ds(off, ch)])
            return carry

        lax.fori_loop(0, n_ch, body, 0)

    return gather_kernel(idx_flat, table)


def _mlp_compute(od, emb_ref, w1b_ref, b1b_ref, w2_ref, b2_ref, out_ref):
    emb = emb_ref[...]
    h = jnp.dot(emb, w1b_ref[...], preferred_element_type=jnp.float32)
    h = jnp.maximum(h + b1b_ref[...], 0.0)
    hb = h.astype(jnp.bfloat16)
    hb = hb.reshape(R, od)
    out_ref[...] = (
        jnp.dot(hb, w2_ref[...], preferred_element_type=jnp.float32)
        + b2_ref[...]
    )


def _mlp_body(od, out_in_ref, emb_ref, w1b_ref, b1b_ref, w2_ref, b2_ref,
              out_ref):
    del out_in_ref
    _mlp_compute(od, emb_ref, w1b_ref, b1b_ref, w2_ref, b2_ref, out_ref)


def _mlp_chunk(out_buf, emb128, w1big, b1big, W2b, b2big, blk_off, n_total):
    od = W2b.shape[1]
    nblk = emb128.shape[0] // (R // 4)
    return pl.pallas_call(
        functools.partial(_mlp_body, od),
        grid=(nblk,),
        in_specs=[
            pl.BlockSpec(memory_space=pl.ANY),
            pl.BlockSpec((R // 4, 128), lambda i: (i, 0)),
            pl.BlockSpec((128, 4 * od), lambda i: (0, 0)),
            pl.BlockSpec((1, 4 * od), lambda i: (0, 0)),
            pl.BlockSpec((od, od), lambda i: (0, 0)),
            pl.BlockSpec((1, od), lambda i: (0, 0)),
        ],
        out_specs=pl.BlockSpec((R, od), lambda i, _o=blk_off: (i + _o, 0)),
        out_shape=jax.ShapeDtypeStruct((n_total, od), jnp.float32),
        input_output_aliases={0: 0},
    )(out_buf, emb128, w1big, b1big, W2b, b2big)


def kernel(action_ids, table, W1, b1, W2, b2):
    B, L = action_ids.shape
    od = W2.shape[1]
    n = B * L
    nc = n // C
    idx = action_ids.reshape(-1).astype(jnp.int32)

    eye = jnp.eye(4, dtype=W1.dtype)
    w1big = jnp.einsum("pq,do->pdqo", eye, W1).reshape(128, 4 * od)
    b1big = jnp.tile(b1, 4).reshape(1, 4 * od)
    W2b = W2.astype(jnp.bfloat16)
    b2big = b2.reshape(1, od)

    embs = [_gather_sc(idx, table, c * nc, nc) for c in range(C)]

    blk_per_chunk = nc // R
    out = None
    for c in range(C):
        emb128 = embs[c].reshape(nc // 4, 128)
        if out is None:
            out = pl.pallas_call(
                functools.partial(_mlp_compute, od),
                grid=(blk_per_chunk,),
                in_specs=[
                    pl.BlockSpec((R // 4, 128), lambda i: (i, 0)),
                    pl.BlockSpec((128, 4 * od), lambda i: (0, 0)),
                    pl.BlockSpec((1, 4 * od), lambda i: (0, 0)),
                    pl.BlockSpec((od, od), lambda i: (0, 0)),
                    pl.BlockSpec((1, od), lambda i: (0, 0)),
                ],
                out_specs=pl.BlockSpec((R, od), lambda i: (i, 0)),
                out_shape=jax.ShapeDtypeStruct((n, od), jnp.float32),
            )(emb128, w1big, b1big, W2b, b2big)
        else:
            out = _mlp_chunk(out, emb128, w1big, b1big, W2b, b2big,
                             c * blk_per_chunk, n)
    return out.reshape(B, L, od)


# speedup vs baseline: 1.0187x; 1.0187x over previous
import functools

import jax
import jax.numpy as jnp
from jax import lax
from jax.experimental import pallas as pl
from jax.experimental.pallas import tpu as pltpu
from jax.experimental.pallas import tpu_sc as plsc

NC, NS = 2, 16
NW = NC * NS
C = 2
R = 16384


def _gather_sc(idx_flat, table, base_row, rows):
    d = table.shape[1]
    per_w = rows // NW
    ch = next(c for c in range(min(per_w, 1280), 0, -1)
              if per_w % c == 0 and c % 8 == 0)
    n_ch = per_w // ch
    mesh = plsc.VectorSubcoreMesh(core_axis_name="c", subcore_axis_name="s")

    @functools.partial(
        pl.kernel,
        mesh=mesh,
        out_type=jax.ShapeDtypeStruct((rows, d), jnp.float32),
        scratch_types=[
            pltpu.VMEM((ch,), jnp.int32),
            pltpu.VMEM((ch, d), jnp.float32),
            pltpu.SemaphoreType.DMA,
        ],
        compiler_params=pltpu.CompilerParams(use_tc_tiling_on_sc=False),
    )
    def gather_kernel(idx_hbm, table_hbm, out_hbm, idx_v, rows_v, sem):
        wid = lax.axis_index("s") * NC + lax.axis_index("c")
        base = wid * per_w

        def body(i, carry):
            off = base + i * ch
            pltpu.sync_copy(idx_hbm.at[pl.ds(base_row + off, ch)], idx_v)
            pltpu.async_copy(table_hbm.at[idx_v], rows_v, sem).wait()
            pltpu.sync_copy(rows_v, out_hbm.at[pl.ds(off, ch)])
            return carry

        lax.fori_loop(0, n_ch, body, 0)

    return gather_kernel(idx_flat, table)


def _mlp_compute(od, emb_ref, w1b_ref, b1b_ref, w2_ref, b2_ref, out_ref):
    emb = emb_ref[...]
    h = jnp.dot(emb, w1b_ref[...], preferred_element_type=jnp.float32)
    h = jnp.maximum(h + b1b_ref[...], 0.0)
    hb = h.astype(jnp.bfloat16)
    hb = hb.reshape(R, od)
    out_ref[...] = (
        jnp.dot(hb, w2_ref[...], preferred_element_type=jnp.float32)
        + b2_ref[...]
    )


def _mlp_body(od, out_in_ref, emb_ref, w1b_ref, b1b_ref, w2_ref, b2_ref,
              out_ref):
    del out_in_ref
    _mlp_compute(od, emb_ref, w1b_ref, b1b_ref, w2_ref, b2_ref, out_ref)


def _mlp_chunk(out_buf, emb128, w1big, b1big, W2b, b2big, blk_off, n_total):
    od = W2b.shape[1]
    nblk = emb128.shape[0] // (R // 4)
    return pl.pallas_call(
        functools.partial(_mlp_body, od),
        grid=(nblk,),
        in_specs=[
            pl.BlockSpec(memory_space=pl.ANY),
            pl.BlockSpec((R // 4, 128), lambda i: (i, 0)),
            pl.BlockSpec((128, 4 * od), lambda i: (0, 0)),
            pl.BlockSpec((1, 4 * od), lambda i: (0, 0)),
            pl.BlockSpec((od, od), lambda i: (0, 0)),
            pl.BlockSpec((1, od), lambda i: (0, 0)),
        ],
        out_specs=pl.BlockSpec((R, od), lambda i, _o=blk_off: (i + _o, 0)),
        out_shape=jax.ShapeDtypeStruct((n_total, od), jnp.float32),
        input_output_aliases={0: 0},
    )(out_buf, emb128, w1big, b1big, W2b, b2big)


def kernel(action_ids, table, W1, b1, W2, b2):
    B, L = action_ids.shape
    od = W2.shape[1]
    n = B * L
    idx = action_ids.reshape(-1).astype(jnp.int32)
    chunk_rows = [12 * R, n - 12 * R]

    eye = jnp.eye(4, dtype=W1.dtype)
    w1big = jnp.einsum("pq,do->pdqo", eye, W1).reshape(128, 4 * od)
    b1big = jnp.tile(b1, 4).reshape(1, 4 * od)
    W2b = W2.astype(jnp.bfloat16)
    b2big = b2.reshape(1, od)

    bases = [0, chunk_rows[0]]
    embs = [_gather_sc(idx, table, bases[c], chunk_rows[c]) for c in range(C)]

    out = None
    for c in range(C):
        nc = chunk_rows[c]
        blk_per_chunk = nc // R
        emb128 = embs[c].reshape(nc // 4, 128)
        if out is None:
            out = pl.pallas_call(
                functools.partial(_mlp_compute, od),
                grid=(blk_per_chunk,),
                in_specs=[
                    pl.BlockSpec((R // 4, 128), lambda i: (i, 0)),
                    pl.BlockSpec((128, 4 * od), lambda i: (0, 0)),
                    pl.BlockSpec((1, 4 * od), lambda i: (0, 0)),
                    pl.BlockSpec((od, od), lambda i: (0, 0)),
                    pl.BlockSpec((1, od), lambda i: (0, 0)),
                ],
                out_specs=pl.BlockSpec((R, od), lambda i: (i, 0)),
                out_shape=jax.ShapeDtypeStruct((n, od), jnp.float32),
            )(emb128, w1big, b1big, W2b, b2big)
        else:
            out = _mlp_chunk(out, emb128, w1big, b1big, W2b, b2big,
                             bases[c] // R, n)
    return out.reshape(B, L, od)


# speedup vs baseline: 1.0225x; 1.0037x over previous
import functools

import jax
import jax.numpy as jnp
from jax import lax
from jax.experimental import pallas as pl
from jax.experimental.pallas import tpu as pltpu
from jax.experimental.pallas import tpu_sc as plsc

NC, NS = 2, 16
NW = NC * NS
C = 2
R = 16384


def _gather_sc(idx_flat, table, base_row, rows):
    d = table.shape[1]
    per_w = rows // NW
    ch = next(c for c in range(min(per_w, 1280), 0, -1)
              if per_w % c == 0 and c % 8 == 0)
    n_ch = per_w // ch
    mesh = plsc.VectorSubcoreMesh(core_axis_name="c", subcore_axis_name="s")

    @functools.partial(
        pl.kernel,
        mesh=mesh,
        out_type=jax.ShapeDtypeStruct((rows, d), jnp.float32),
        scratch_types=[
            pltpu.VMEM((ch,), jnp.int32),
            pltpu.VMEM((ch, d), jnp.float32),
            pltpu.SemaphoreType.DMA,
        ],
        compiler_params=pltpu.CompilerParams(use_tc_tiling_on_sc=False),
    )
    def gather_kernel(idx_hbm, table_hbm, out_hbm, idx_v, rows_v, sem):
        wid = lax.axis_index("s") * NC + lax.axis_index("c")
        base = wid * per_w

        def body(i, carry):
            off = base + i * ch
            pltpu.sync_copy(idx_hbm.at[pl.ds(base_row + off, ch)], idx_v)
            pltpu.async_copy(table_hbm.at[idx_v], rows_v, sem).wait()
            pltpu.sync_copy(rows_v, out_hbm.at[pl.ds(off, ch)])
            return carry

        lax.fori_loop(0, n_ch, body, 0)

    return gather_kernel(idx_flat, table)


def _mlp_compute(od, emb_ref, w1b_ref, b1b_ref, w2_ref, b2_ref, out_ref):
    emb = emb_ref[...]
    h = jnp.dot(emb, w1b_ref[...], preferred_element_type=jnp.float32)
    h = jnp.maximum(h + b1b_ref[...], 0.0)
    hb = h.astype(jnp.bfloat16)
    hb = hb.reshape(R, od)
    out_ref[...] = (
        jnp.dot(hb, w2_ref[...], preferred_element_type=jnp.float32)
        + b2_ref[...]
    )


def _mlp_body(od, out_in_ref, emb_ref, w1b_ref, b1b_ref, w2_ref, b2_ref,
              out_ref):
    del out_in_ref
    _mlp_compute(od, emb_ref, w1b_ref, b1b_ref, w2_ref, b2_ref, out_ref)


def _mlp_chunk(out_buf, emb128, w1big, b1big, W2b, b2big, blk_off, n_total):
    od = W2b.shape[1]
    nblk = emb128.shape[0] // (R // 4)
    return pl.pallas_call(
        functools.partial(_mlp_body, od),
        grid=(nblk,),
        in_specs=[
            pl.BlockSpec(memory_space=pl.ANY),
            pl.BlockSpec((R // 4, 128), lambda i: (i, 0)),
            pl.BlockSpec((128, 4 * od), lambda i: (0, 0)),
            pl.BlockSpec((1, 4 * od), lambda i: (0, 0)),
            pl.BlockSpec((od, od), lambda i: (0, 0)),
            pl.BlockSpec((1, od), lambda i: (0, 0)),
        ],
        out_specs=pl.BlockSpec((R, od), lambda i, _o=blk_off: (i + _o, 0)),
        out_shape=jax.ShapeDtypeStruct((n_total, od), jnp.float32),
        input_output_aliases={0: 0},
    )(out_buf, emb128, w1big, b1big, W2b, b2big)


def kernel(action_ids, table, W1, b1, W2, b2):
    B, L = action_ids.shape
    od = W2.shape[1]
    n = B * L
    idx = action_ids.reshape(-1).astype(jnp.int32)
    chunk_rows = [8 * R, n - 8 * R]

    eye = jnp.eye(4, dtype=W1.dtype)
    w1big = jnp.einsum("pq,do->pdqo", eye, W1).reshape(128, 4 * od)
    b1big = jnp.tile(b1, 4).reshape(1, 4 * od)
    W2b = W2.astype(jnp.bfloat16)
    b2big = b2.reshape(1, od)

    bases = [0, chunk_rows[0]]
    embs = [_gather_sc(idx, table, bases[c], chunk_rows[c]) for c in range(C)]

    out = None
    for c in range(C):
        nc = chunk_rows[c]
        blk_per_chunk = nc // R
        emb128 = embs[c].reshape(nc // 4, 128)
        if out is None:
            out = pl.pallas_call(
                functools.partial(_mlp_compute, od),
                grid=(blk_per_chunk,),
                in_specs=[
                    pl.BlockSpec((R // 4, 128), lambda i: (i, 0)),
                    pl.BlockSpec((128, 4 * od), lambda i: (0, 0)),
                    pl.BlockSpec((1, 4 * od), lambda i: (0, 0)),
                    pl.BlockSpec((od, od), lambda i: (0, 0)),
                    pl.BlockSpec((1, od), lambda i: (0, 0)),
                ],
                out_specs=pl.BlockSpec((R, od), lambda i: (i, 0)),
                out_shape=jax.ShapeDtypeStruct((n, od), jnp.float32),
            )(emb128, w1big, b1big, W2b, b2big)
        else:
            out = _mlp_chunk(out, emb128, w1big, b1big, W2b, b2big,
                             bases[c] // R, n)
    return out.reshape(B, L, od)
